# baseline (speedup 1.0000x reference)
import jax
import jax.numpy as jnp
from jax import lax
from jax.experimental import pallas as pl
from jax.experimental.pallas import tpu as pltpu
from jax.experimental.pallas import tpu_sc as plsc

_N = 4096
_DM = 1024
_DFF = 2048
_E = 8
_K = 2

_BT = 256
_NT = (_N * _K) // _BT + _E
_R = _NT * _BT
_FB = 512
_NF = _DFF // _FB

_NC = 2
_NS = 16
_NW = _NC * _NS

_ROWS_W = _R // _NW
_GCH = 16
_NCH = _ROWS_W // _GCH
_GNB = 4
_TOK_W = _N // _NW
_CCH = 16
_NCC = _TOK_W // _CCH


def _route(x, w_gate):
    logitsT = lax.dot_general(w_gate, x, (((0,), (1,)), ((), ())))
    erange = jnp.arange(_E, dtype=jnp.int32)[:, None]
    v1 = jnp.max(logitsT, axis=0)
    m1 = jnp.argmax(logitsT, axis=0).astype(jnp.int32)
    neg = jnp.where(erange == m1[None, :], -jnp.inf, logitsT)
    v2 = jnp.max(neg, axis=0)
    m2 = jnp.argmax(neg, axis=0).astype(jnp.int32)
    e2 = jnp.exp(v2 - v1)
    g1 = 1.0 / (1.0 + e2)
    g2 = e2 * g1
    e_lo = jnp.minimum(m1, m2)
    e_hi = jnp.maximum(m1, m2)
    swap = m1 != e_lo
    g_lo = jnp.where(swap, g2, g1)
    g_hi = jnp.where(swap, g1, g2)
    oh_lo = erange == e_lo[None, :]
    oh_hi = erange == e_hi[None, :]
    ind = (oh_lo | oh_hi).astype(jnp.int32)
    csum = jnp.cumsum(ind, axis=1)
    counts = csum[:, -1]
    excl = csum - ind
    padded = ((counts + _BT - 1) // _BT) * _BT
    pad_start = jnp.concatenate(
        [jnp.zeros((1,), jnp.int32), jnp.cumsum(padded).astype(jnp.int32)])
    starts = pad_start[:_E, None]
    pos0 = jnp.sum(jnp.where(oh_lo, starts + excl, 0), axis=0)
    pos1 = jnp.sum(jnp.where(oh_hi, starts + excl, 0), axis=0)
    tok = jnp.arange(_N, dtype=jnp.int32)
    row_ids = jnp.zeros((_R,), jnp.int32).at[pos0].set(tok).at[pos1].set(tok)
    gate_pad = (jnp.zeros((_R,), jnp.float32)
                .at[pos0].set(g_lo).at[pos1].set(g_hi))
    tile_base = jnp.arange(_NT, dtype=jnp.int32) * _BT
    ends = pad_start[1:]
    eid = jnp.sum((ends[None, :] <= tile_base[:, None]).astype(jnp.int32),
                  axis=1)
    eid = jnp.minimum(eid, _E - 1)
    eid = jnp.where(tile_base < pad_start[_E], eid, -1).astype(jnp.int32)
    return row_ids, gate_pad, eid, pos0, pos1


def _gather_body(x_hbm, ids_hbm, out_hbm, idx_v, rows0, rows1, rows2, rows3,
                 sg0, sg1, sg2, sg3, ss0, ss1, ss2, ss3):
    wid = lax.axis_index("s") * _NC + lax.axis_index("c")
    pltpu.sync_copy(ids_hbm.at[wid], idx_v)
    bufs = (rows0, rows1, rows2, rows3)
    sgs = (sg0, sg1, sg2, sg3)
    sss = (ss0, ss1, ss2, ss3)
    gets = [None] * _NCH
    puts = [None] * _NCH
    for j in range(_GNB - 1):
        gets[j] = pltpu.async_copy(x_hbm.at[idx_v.at[j]], bufs[j], sgs[j])
    for i in range(_NCH):
        b = i % _GNB
        gets[i].wait()
        puts[i] = pltpu.async_copy(
            bufs[b], out_hbm.at[pl.ds(wid * _ROWS_W + i * _GCH, _GCH)],
            sss[b])
        j = i + _GNB - 1
        if j < _NCH:
            jb = j % _GNB
            if j >= _GNB:
                puts[j - _GNB].wait()
            gets[j] = pltpu.async_copy(
                x_hbm.at[idx_v.at[j]], bufs[jb], sgs[jb])
    for i in range(_NCH - _GNB, _NCH):
        puts[i].wait()


def _add_rows(dst, src):
    def row(rr, carry):
        for u in range(_DM // 16):
            sl = pl.ds(u * 16, 16)
            dst[rr, sl] = dst[rr, sl] + src[rr, sl]
        return carry

    lax.fori_loop(0, _CCH, row, 0)


def _combine_body(yg_hbm, p0_hbm, p1_hbm, out_hbm, i0_v, i1_v,
                  r0a, r1a, r0b, r1b, sa0, sa1, sb0, sb1, so0, so1):
    wid = lax.axis_index("s") * _NC + lax.axis_index("c")
    pltpu.sync_copy(p0_hbm.at[wid], i0_v)
    pltpu.sync_copy(p1_hbm.at[wid], i1_v)
    r0s = (r0a, r0b)
    r1s = (r1a, r1b)
    s0s = (sa0, sb0)
    s1s = (sa1, sb1)
    sos = (so0, so1)
    g0 = [None] * _NCC
    g1 = [None] * _NCC
    puts = [None] * _NCC
    g0[0] = pltpu.async_copy(yg_hbm.at[i0_v.at[0]], r0a, sa0)
    g1[0] = pltpu.async_copy(yg_hbm.at[i1_v.at[0]], r1a, sa1)
    for c in range(_NCC):
        b = c % 2
        g0[c].wait()
        g1[c].wait()
        if c + 1 < _NCC:
            if c >= 1:
                puts[c - 1].wait()
            g0[c + 1] = pltpu.async_copy(
                yg_hbm.at[i0_v.at[c + 1]], r0s[1 - b], s0s[1 - b])
            g1[c + 1] = pltpu.async_copy(
                yg_hbm.at[i1_v.at[c + 1]], r1s[1 - b], s1s[1 - b])
        _add_rows(r0s[b], r1s[b])
        puts[c] = pltpu.async_copy(
            r0s[b], out_hbm.at[pl.ds(wid * _TOK_W + c * _CCH, _CCH)],
            sos[b])
    puts[_NCC - 2].wait()
    puts[_NCC - 1].wait()


_sc_calls_cache = None


def _sc_calls():
    global _sc_calls_cache
    if _sc_calls_cache is None:
        mesh = plsc.VectorSubcoreMesh(
            core_axis_name="c", subcore_axis_name="s", num_cores=_NC)
        gather = pl.kernel(
            _gather_body, mesh=mesh,
            out_type=jax.ShapeDtypeStruct((_R, 8, 128), jnp.float32),
            scratch_types=[
                pltpu.VMEM((_NCH, _GCH), jnp.int32),
            ] + [pltpu.VMEM((_GCH, 8, 128), jnp.float32)] * _GNB
              + [pltpu.SemaphoreType.DMA] * (2 * _GNB),
        )
        combine = pl.kernel(
            _combine_body, mesh=mesh,
            out_type=jax.ShapeDtypeStruct((_N, _DM), jnp.float32),
            scratch_types=[
                pltpu.VMEM((_NCC, _CCH), jnp.int32),
                pltpu.VMEM((_NCC, _CCH), jnp.int32),
                pltpu.VMEM((_CCH, _DM), jnp.float32),
                pltpu.VMEM((_CCH, _DM), jnp.float32),
                pltpu.VMEM((_CCH, _DM), jnp.float32),
                pltpu.VMEM((_CCH, _DM), jnp.float32),
            ] + [pltpu.SemaphoreType.DMA] * 6,
        )
        _sc_calls_cache = (gather, combine)
    return _sc_calls_cache


def _mlp_body(eid_ref, xg_ref, w1_ref, b1_ref, w2_ref, b2_ref, gate_ref,
              out_ref):
    t = pl.program_id(0)
    f = pl.program_id(1)
    valid = eid_ref[t] >= 0

    @pl.when(valid)
    def _():
        h = jnp.dot(xg_ref[...], w1_ref[0],
                    preferred_element_type=jnp.float32)
        h = jnp.maximum(h + b1_ref[0], 0.0)
        contrib = jnp.dot(h, w2_ref[0], preferred_element_type=jnp.float32)

        @pl.when(f == 0)
        def _():
            out_ref[...] = contrib

        @pl.when(f != 0)
        def _():
            out_ref[...] += contrib

        @pl.when(f == _NF - 1)
        def _():
            out_ref[...] = (out_ref[...] + b2_ref[0]) * gate_ref[...]


def _eix(e_ref, t):
    return jnp.maximum(e_ref[t], 0)


_mlp_call = pl.pallas_call(
    _mlp_body,
    grid_spec=pltpu.PrefetchScalarGridSpec(
        num_scalar_prefetch=1,
        grid=(_NT, _NF),
        in_specs=[
            pl.BlockSpec((_BT, _DM), lambda t, f, e: (t, 0)),
            pl.BlockSpec((1, _DM, _FB), lambda t, f, e: (_eix(e, t), 0, f)),
            pl.BlockSpec((1, 1, _FB), lambda t, f, e: (_eix(e, t), 0, f)),
            pl.BlockSpec((1, _FB, _DM), lambda t, f, e: (_eix(e, t), f, 0)),
            pl.BlockSpec((1, 1, _DM), lambda t, f, e: (_eix(e, t), 0, 0)),
            pl.BlockSpec((_BT, 1), lambda t, f, e: (t, 0)),
        ],
        out_specs=pl.BlockSpec((_BT, _DM), lambda t, f, e: (t, 0)),
    ),
    out_shape=jax.ShapeDtypeStruct((_R, _DM), jnp.float32),
)


def kernel(x, w_gate, W1, b1, W2, b2):
    _gather_rows, _combine = _sc_calls()
    row_ids, gate_pad, eid, pos0, pos1 = _route(x, w_gate)
    xg = _gather_rows(x.reshape(_N, 8, 128),
                      row_ids.reshape(_NW, _NCH, _GCH)).reshape(_R, _DM)
    yg = _mlp_call(
        eid, xg,
        W1, b1.reshape(_E, 1, _DFF), W2, b2.reshape(_E, 1, _DM),
        gate_pad.reshape(_R, 1))
    out = _combine(
        yg,
        pos0.reshape(_NW, _NCC, _CCH),
        pos1.reshape(_NW, _NCC, _CCH))
    return out


# speedup vs baseline: 1.0847x; 1.0847x over previous
import jax
import jax.numpy as jnp
from jax import lax
from jax.experimental import pallas as pl
from jax.experimental.pallas import tpu as pltpu
from jax.experimental.pallas import tpu_sc as plsc

_N = 4096
_DM = 1024
_DFF = 2048
_E = 8
_K = 2

_BT = 256
_NT = (_N * _K) // _BT + _E
_R = _NT * _BT
_FB = 512
_NF = _DFF // _FB

_NC = 2
_NS = 16
_NW = _NC * _NS

_ROWS_W = _R // _NW
_GCH = 16
_NCH = _ROWS_W // _GCH
_GNB = 4
_TOK_W = _N // _NW
_CCH = 16
_NCC = _TOK_W // _CCH


def _route(x, w_gate):
    logitsT = lax.dot_general(w_gate, x, (((0,), (1,)), ((), ())))
    erange = jnp.arange(_E, dtype=jnp.int32)[:, None]
    v1 = jnp.max(logitsT, axis=0)
    m1 = jnp.argmax(logitsT, axis=0).astype(jnp.int32)
    neg = jnp.where(erange == m1[None, :], -jnp.inf, logitsT)
    v2 = jnp.max(neg, axis=0)
    m2 = jnp.argmax(neg, axis=0).astype(jnp.int32)
    e2 = jnp.exp(v2 - v1)
    g1 = 1.0 / (1.0 + e2)
    g2 = e2 * g1
    e_lo = jnp.minimum(m1, m2)
    e_hi = jnp.maximum(m1, m2)
    swap = m1 != e_lo
    g_lo = jnp.where(swap, g2, g1)
    g_hi = jnp.where(swap, g1, g2)
    oh_lo = erange == e_lo[None, :]
    oh_hi = erange == e_hi[None, :]
    ind = (oh_lo | oh_hi).astype(jnp.int32)
    csum = jnp.cumsum(ind, axis=1)
    counts = csum[:, -1]
    excl = csum - ind
    padded = ((counts + _BT - 1) // _BT) * _BT
    pad_start = jnp.concatenate(
        [jnp.zeros((1,), jnp.int32), jnp.cumsum(padded).astype(jnp.int32)])
    starts = pad_start[:_E, None]
    pos0 = jnp.sum(jnp.where(oh_lo, starts + excl, 0), axis=0)
    pos1 = jnp.sum(jnp.where(oh_hi, starts + excl, 0), axis=0)
    tok = jnp.arange(_N, dtype=jnp.int32)
    row_ids = jnp.zeros((_R,), jnp.int32).at[pos0].set(tok).at[pos1].set(tok)
    gate_pad = (jnp.zeros((_R,), jnp.float32)
                .at[pos0].set(g_lo).at[pos1].set(g_hi))
    tile_base = jnp.arange(_NT, dtype=jnp.int32) * _BT
    ends = pad_start[1:]
    eid = jnp.sum((ends[None, :] <= tile_base[:, None]).astype(jnp.int32),
                  axis=1)
    eid = jnp.minimum(eid, _E - 1)
    eid = jnp.where(tile_base < pad_start[_E], eid, -1).astype(jnp.int32)
    return row_ids, gate_pad, eid, pos0, pos1


def _gather_body(x_hbm, ids_hbm, out_hbm, idx_v, rows0, rows1, rows2, rows3,
                 sg0, sg1, sg2, sg3, ss0, ss1, ss2, ss3):
    wid = lax.axis_index("s") * _NC + lax.axis_index("c")
    pltpu.sync_copy(ids_hbm.at[wid], idx_v)
    bufs = (rows0, rows1, rows2, rows3)
    sgs = (sg0, sg1, sg2, sg3)
    sss = (ss0, ss1, ss2, ss3)
    gets = [None] * _NCH
    puts = [None] * _NCH
    for j in range(_GNB - 1):
        gets[j] = pltpu.async_copy(x_hbm.at[idx_v[j, :]], bufs[j], sgs[j])
    for i in range(_NCH):
        b = i % _GNB
        gets[i].wait()
        puts[i] = pltpu.async_copy(
            bufs[b], out_hbm.at[pl.ds(wid * _ROWS_W + i * _GCH, _GCH)],
            sss[b])
        j = i + _GNB - 1
        if j < _NCH:
            jb = j % _GNB
            if j >= _GNB:
                puts[j - _GNB].wait()
            gets[j] = pltpu.async_copy(
                x_hbm.at[idx_v[j, :]], bufs[jb], sgs[jb])
    for i in range(_NCH - _GNB, _NCH):
        puts[i].wait()


def _add_rows(dst, src):
    def row(rr, carry):
        for u in range(_DM // 16):
            sl = pl.ds(u * 16, 16)
            dst[rr, sl] = dst[rr, sl] + src[rr, sl]
        return carry

    lax.fori_loop(0, _CCH, row, 0)


def _combine_body(yg_hbm, p0_hbm, p1_hbm, out_hbm, i0_v, i1_v,
                  r0a, r1a, r0b, r1b, sa0, sa1, sb0, sb1, so0, so1):
    wid = lax.axis_index("s") * _NC + lax.axis_index("c")
    pltpu.sync_copy(p0_hbm.at[wid], i0_v)
    pltpu.sync_copy(p1_hbm.at[wid], i1_v)
    r0s = (r0a, r0b)
    r1s = (r1a, r1b)
    s0s = (sa0, sb0)
    s1s = (sa1, sb1)
    sos = (so0, so1)
    g0 = [None] * _NCC
    g1 = [None] * _NCC
    puts = [None] * _NCC
    g0[0] = pltpu.async_copy(yg_hbm.at[i0_v.at[0]], r0a, sa0)
    g1[0] = pltpu.async_copy(yg_hbm.at[i1_v.at[0]], r1a, sa1)
    for c in range(_NCC):
        b = c % 2
        g0[c].wait()
        g1[c].wait()
        if c + 1 < _NCC:
            if c >= 1:
                puts[c - 1].wait()
            g0[c + 1] = pltpu.async_copy(
                yg_hbm.at[i0_v.at[c + 1]], r0s[1 - b], s0s[1 - b])
            g1[c + 1] = pltpu.async_copy(
                yg_hbm.at[i1_v.at[c + 1]], r1s[1 - b], s1s[1 - b])
        _add_rows(r0s[b], r1s[b])
        puts[c] = pltpu.async_copy(
            r0s[b], out_hbm.at[pl.ds(wid * _TOK_W + c * _CCH, _CCH)],
            sos[b])
    puts[_NCC - 2].wait()
    puts[_NCC - 1].wait()


_sc_calls_cache = None


def _sc_calls():
    global _sc_calls_cache
    if _sc_calls_cache is None:
        mesh = plsc.VectorSubcoreMesh(
            core_axis_name="c", subcore_axis_name="s", num_cores=_NC)
        gather = pl.kernel(
            _gather_body, mesh=mesh,
            out_type=jax.ShapeDtypeStruct((_R, _DM), jnp.float32),
            scratch_types=[
                pltpu.VMEM((_NCH, _GCH), jnp.int32),
            ] + [pltpu.VMEM((_GCH, _DM), jnp.float32)] * _GNB
              + [pltpu.SemaphoreType.DMA] * (2 * _GNB),
        )
        combine = pl.kernel(
            _combine_body, mesh=mesh,
            out_type=jax.ShapeDtypeStruct((_N, _DM), jnp.float32),
            scratch_types=[
                pltpu.VMEM((_NCC, _CCH), jnp.int32),
                pltpu.VMEM((_NCC, _CCH), jnp.int32),
                pltpu.VMEM((_CCH, _DM), jnp.float32),
                pltpu.VMEM((_CCH, _DM), jnp.float32),
                pltpu.VMEM((_CCH, _DM), jnp.float32),
                pltpu.VMEM((_CCH, _DM), jnp.float32),
            ] + [pltpu.SemaphoreType.DMA] * 6,
        )
        _sc_calls_cache = (gather, combine)
    return _sc_calls_cache


def _mlp_body(eid_ref, xg_ref, w1_ref, b1_ref, w2_ref, b2_ref, gate_ref,
              out_ref):
    t = pl.program_id(0)
    f = pl.program_id(1)
    valid = eid_ref[t] >= 0

    @pl.when(valid)
    def _():
        h = jnp.dot(xg_ref[...], w1_ref[0],
                    preferred_element_type=jnp.float32)
        h = jnp.maximum(h + b1_ref[0], 0.0)
        contrib = jnp.dot(h, w2_ref[0], preferred_element_type=jnp.float32)

        @pl.when(f == 0)
        def _():
            out_ref[...] = contrib

        @pl.when(f != 0)
        def _():
            out_ref[...] += contrib

        @pl.when(f == _NF - 1)
        def _():
            out_ref[...] = (out_ref[...] + b2_ref[0]) * gate_ref[...]


def _eix(e_ref, t):
    return jnp.maximum(e_ref[t], 0)


_mlp_call = pl.pallas_call(
    _mlp_body,
    grid_spec=pltpu.PrefetchScalarGridSpec(
        num_scalar_prefetch=1,
        grid=(_NT, _NF),
        in_specs=[
            pl.BlockSpec((_BT, _DM), lambda t, f, e: (t, 0)),
            pl.BlockSpec((1, _DM, _FB), lambda t, f, e: (_eix(e, t), 0, f)),
            pl.BlockSpec((1, 1, _FB), lambda t, f, e: (_eix(e, t), 0, f)),
            pl.BlockSpec((1, _FB, _DM), lambda t, f, e: (_eix(e, t), f, 0)),
            pl.BlockSpec((1, 1, _DM), lambda t, f, e: (_eix(e, t), 0, 0)),
            pl.BlockSpec((_BT, 1), lambda t, f, e: (t, 0)),
        ],
        out_specs=pl.BlockSpec((_BT, _DM), lambda t, f, e: (t, 0)),
    ),
    out_shape=jax.ShapeDtypeStruct((_R, _DM), jnp.float32),
)


def kernel(x, w_gate, W1, b1, W2, b2):
    _gather_rows, _combine = _sc_calls()
    row_ids, gate_pad, eid, pos0, pos1 = _route(x, w_gate)
    xg = _gather_rows(x, row_ids.reshape(_NW, _NCH, _GCH))
    yg = _mlp_call(
        eid, xg,
        W1, b1.reshape(_E, 1, _DFF), W2, b2.reshape(_E, 1, _DM),
        gate_pad.reshape(_R, 1))
    out = _combine(
        yg,
        pos0.reshape(_NW, _NCC, _CCH),
        pos1.reshape(_NW, _NCC, _CCH))
    return out


# speedup vs baseline: 1.5671x; 1.4448x over previous
import jax
import jax.numpy as jnp
from jax import lax
from jax.experimental import pallas as pl
from jax.experimental.pallas import tpu as pltpu
from jax.experimental.pallas import tpu_sc as plsc

_N = 4096
_DM = 1024
_DFF = 2048
_E = 8
_K = 2

_BT = 256
_NT = (_N * _K) // _BT + _E
_R = _NT * _BT
_FB = 1024
_NF = _DFF // _FB

_NC = 2
_NS = 16
_NW = _NC * _NS

_ROWS_W = _R // _NW
_GCH = 16
_NCH = _ROWS_W // _GCH
_GNB = 4
_TOK_W = _N // _NW
_CCH = 16
_NCC = _TOK_W // _CCH


def _route(x, w_gate):
    logitsT = lax.dot_general(w_gate, x, (((0,), (1,)), ((), ())))
    erange = jnp.arange(_E, dtype=jnp.int32)[:, None]
    v1 = jnp.max(logitsT, axis=0)
    m1 = jnp.argmax(logitsT, axis=0).astype(jnp.int32)
    neg = jnp.where(erange == m1[None, :], -jnp.inf, logitsT)
    v2 = jnp.max(neg, axis=0)
    m2 = jnp.argmax(neg, axis=0).astype(jnp.int32)
    e2 = jnp.exp(v2 - v1)
    g1 = 1.0 / (1.0 + e2)
    g2 = e2 * g1
    e_lo = jnp.minimum(m1, m2)
    e_hi = jnp.maximum(m1, m2)
    swap = m1 != e_lo
    g_lo = jnp.where(swap, g2, g1)
    g_hi = jnp.where(swap, g1, g2)
    oh_lo = erange == e_lo[None, :]
    oh_hi = erange == e_hi[None, :]
    ind = (oh_lo | oh_hi).astype(jnp.int32)
    csum = jnp.cumsum(ind, axis=1)
    counts = csum[:, -1]
    excl = csum - ind
    padded = ((counts + _BT - 1) // _BT) * _BT
    pad_start = jnp.concatenate(
        [jnp.zeros((1,), jnp.int32), jnp.cumsum(padded).astype(jnp.int32)])
    starts = pad_start[:_E, None]
    pos0 = jnp.sum(jnp.where(oh_lo, starts + excl, 0), axis=0)
    pos1 = jnp.sum(jnp.where(oh_hi, starts + excl, 0), axis=0)
    tok = jnp.arange(_N, dtype=jnp.int32)
    base_ids = jnp.arange(_R, dtype=jnp.int32) % _N
    row_ids = base_ids.at[pos0].set(tok).at[pos1].set(tok)
    gate_pad = (jnp.zeros((_R,), jnp.float32)
                .at[pos0].set(g_lo).at[pos1].set(g_hi))
    tile_base = jnp.arange(_NT, dtype=jnp.int32) * _BT
    ends = pad_start[1:]
    eid = jnp.sum((ends[None, :] <= tile_base[:, None]).astype(jnp.int32),
                  axis=1)
    eid = jnp.minimum(eid, _E - 1)
    eid = jnp.where(tile_base < pad_start[_E], eid, -1).astype(jnp.int32)
    return row_ids, gate_pad, eid, pos0, pos1


def _gather_body(x_hbm, ids_hbm, out_hbm, idx_v, rows0, rows1, rows2, rows3,
                 sg0, sg1, sg2, sg3, ss0, ss1, ss2, ss3):
    wid = lax.axis_index("s") * _NC + lax.axis_index("c")
    pltpu.sync_copy(ids_hbm.at[wid], idx_v)
    bufs = (rows0, rows1, rows2, rows3)
    sgs = (sg0, sg1, sg2, sg3)
    sss = (ss0, ss1, ss2, ss3)
    gets = [None] * _NCH
    puts = [None] * _NCH
    for j in range(_GNB - 1):
        gets[j] = pltpu.async_copy(x_hbm.at[idx_v[j, :]], bufs[j], sgs[j])
    for i in range(_NCH):
        b = i % _GNB
        gets[i].wait()
        puts[i] = pltpu.async_copy(
            bufs[b], out_hbm.at[pl.ds(wid * _ROWS_W + i * _GCH, _GCH)],
            sss[b])
        j = i + _GNB - 1
        if j < _NCH:
            jb = j % _GNB
            if j >= _GNB:
                puts[j - _GNB].wait()
            gets[j] = pltpu.async_copy(
                x_hbm.at[idx_v[j, :]], bufs[jb], sgs[jb])
    for i in range(_NCH - _GNB, _NCH):
        puts[i].wait()


def _add_rows(dst, src):
    def row(rr, carry):
        for u in range(_DM // 16):
            sl = pl.ds(u * 16, 16)
            dst[rr, sl] = dst[rr, sl] + src[rr, sl]
        return carry

    lax.fori_loop(0, _CCH, row, 0)


def _combine_body(yg_hbm, p0_hbm, p1_hbm, out_hbm, i0_v, i1_v,
                  r0a, r1a, r0b, r1b, sa0, sa1, sb0, sb1, so0, so1):
    wid = lax.axis_index("s") * _NC + lax.axis_index("c")
    pltpu.sync_copy(p0_hbm.at[wid], i0_v)
    pltpu.sync_copy(p1_hbm.at[wid], i1_v)
    r0s = (r0a, r0b)
    r1s = (r1a, r1b)
    s0s = (sa0, sb0)
    s1s = (sa1, sb1)
    sos = (so0, so1)
    g0 = [None] * _NCC
    g1 = [None] * _NCC
    puts = [None] * _NCC
    g0[0] = pltpu.async_copy(yg_hbm.at[i0_v.at[0]], r0a, sa0)
    g1[0] = pltpu.async_copy(yg_hbm.at[i1_v.at[0]], r1a, sa1)
    for c in range(_NCC):
        b = c % 2
        g0[c].wait()
        g1[c].wait()
        if c + 1 < _NCC:
            if c >= 1:
                puts[c - 1].wait()
            g0[c + 1] = pltpu.async_copy(
                yg_hbm.at[i0_v.at[c + 1]], r0s[1 - b], s0s[1 - b])
            g1[c + 1] = pltpu.async_copy(
                yg_hbm.at[i1_v.at[c + 1]], r1s[1 - b], s1s[1 - b])
        _add_rows(r0s[b], r1s[b])
        puts[c] = pltpu.async_copy(
            r0s[b], out_hbm.at[pl.ds(wid * _TOK_W + c * _CCH, _CCH)],
            sos[b])
    puts[_NCC - 2].wait()
    puts[_NCC - 1].wait()


_sc_calls_cache = None


def _sc_calls():
    global _sc_calls_cache
    if _sc_calls_cache is None:
        mesh = plsc.VectorSubcoreMesh(
            core_axis_name="c", subcore_axis_name="s", num_cores=_NC)
        gather = pl.kernel(
            _gather_body, mesh=mesh,
            out_type=jax.ShapeDtypeStruct((_R, _DM), jnp.float32),
            scratch_types=[
                pltpu.VMEM((_NCH, _GCH), jnp.int32),
            ] + [pltpu.VMEM((_GCH, _DM), jnp.float32)] * _GNB
              + [pltpu.SemaphoreType.DMA] * (2 * _GNB),
        )
        combine = pl.kernel(
            _combine_body, mesh=mesh,
            out_type=jax.ShapeDtypeStruct((_N, _DM), jnp.float32),
            scratch_types=[
                pltpu.VMEM((_NCC, _CCH), jnp.int32),
                pltpu.VMEM((_NCC, _CCH), jnp.int32),
                pltpu.VMEM((_CCH, _DM), jnp.float32),
                pltpu.VMEM((_CCH, _DM), jnp.float32),
                pltpu.VMEM((_CCH, _DM), jnp.float32),
                pltpu.VMEM((_CCH, _DM), jnp.float32),
            ] + [pltpu.SemaphoreType.DMA] * 6,
        )
        _sc_calls_cache = (gather, combine)
    return _sc_calls_cache


def _mlp_body(eid_ref, xg_ref, w1_ref, b1_ref, w2_ref, b2_ref, gate_ref,
              out_ref):
    t = pl.program_id(0)
    f = pl.program_id(1)
    valid = eid_ref[t] >= 0

    @pl.when(valid)
    def _():
        h = jnp.dot(xg_ref[...], w1_ref[0],
                    preferred_element_type=jnp.float32)
        h = jnp.maximum(h + b1_ref[0], 0.0)
        contrib = jnp.dot(h, w2_ref[0], preferred_element_type=jnp.float32)

        @pl.when(f == 0)
        def _():
            out_ref[...] = contrib

        @pl.when(f != 0)
        def _():
            out_ref[...] += contrib

        @pl.when(f == _NF - 1)
        def _():
            out_ref[...] = (out_ref[...] + b2_ref[0]) * gate_ref[...]


def _eix(e_ref, t):
    return jnp.maximum(e_ref[t], 0)


_mlp_call = pl.pallas_call(
    _mlp_body,
    grid_spec=pltpu.PrefetchScalarGridSpec(
        num_scalar_prefetch=1,
        grid=(_NT, _NF),
        in_specs=[
            pl.BlockSpec((_BT, _DM), lambda t, f, e: (t, 0)),
            pl.BlockSpec((1, _DM, _FB), lambda t, f, e: (_eix(e, t), 0, f)),
            pl.BlockSpec((1, 1, _FB), lambda t, f, e: (_eix(e, t), 0, f)),
            pl.BlockSpec((1, _FB, _DM), lambda t, f, e: (_eix(e, t), f, 0)),
            pl.BlockSpec((1, 1, _DM), lambda t, f, e: (_eix(e, t), 0, 0)),
            pl.BlockSpec((_BT, 1), lambda t, f, e: (t, 0)),
        ],
        out_specs=pl.BlockSpec((_BT, _DM), lambda t, f, e: (t, 0)),
    ),
    out_shape=jax.ShapeDtypeStruct((_R, _DM), jnp.float32),
)


def kernel(x, w_gate, W1, b1, W2, b2):
    _gather_rows, _combine = _sc_calls()
    row_ids, gate_pad, eid, pos0, pos1 = _route(x, w_gate)
    xg = _gather_rows(x, row_ids.reshape(_NW, _NCH, _GCH))
    yg = _mlp_call(
        eid, xg,
        W1, b1.reshape(_E, 1, _DFF), W2, b2.reshape(_E, 1, _DM),
        gate_pad.reshape(_R, 1))
    out = _combine(
        yg,
        pos0.reshape(_NW, _NCC, _CCH),
        pos1.reshape(_NW, _NCC, _CCH))
    return out


# speedup vs baseline: 2.1248x; 1.3558x over previous
import jax
import jax.numpy as jnp
from jax import lax
from jax.experimental import pallas as pl
from jax.experimental.pallas import tpu as pltpu
from jax.experimental.pallas import tpu_sc as plsc

_N = 4096
_DM = 1024
_DFF = 2048
_E = 8
_K = 2

_BT = 256
_NT = (_N * _K) // _BT + _E
_R = _NT * _BT
_FB = 2048
_NF = _DFF // _FB

_NC = 2
_NS = 16
_NW = _NC * _NS

_ROWS_W = _R // _NW
_GCH = 16
_NCH = _ROWS_W // _GCH
_GNB = 4
_TOK_W = _N // _NW
_CCH = 16
_NCC = _TOK_W // _CCH


def _route(x, w_gate):
    logitsT = lax.dot_general(w_gate, x, (((0,), (1,)), ((), ())))
    erange = jnp.arange(_E, dtype=jnp.int32)[:, None]
    v1 = jnp.max(logitsT, axis=0)
    m1 = jnp.argmax(logitsT, axis=0).astype(jnp.int32)
    neg = jnp.where(erange == m1[None, :], -jnp.inf, logitsT)
    v2 = jnp.max(neg, axis=0)
    m2 = jnp.argmax(neg, axis=0).astype(jnp.int32)
    e2 = jnp.exp(v2 - v1)
    g1 = 1.0 / (1.0 + e2)
    g2 = e2 * g1
    e_lo = jnp.minimum(m1, m2)
    e_hi = jnp.maximum(m1, m2)
    swap = m1 != e_lo
    g_lo = jnp.where(swap, g2, g1)
    g_hi = jnp.where(swap, g1, g2)
    oh_lo = erange == e_lo[None, :]
    oh_hi = erange == e_hi[None, :]
    ind = (oh_lo | oh_hi).astype(jnp.int32)
    csum = jnp.cumsum(ind, axis=1)
    counts = csum[:, -1]
    excl = csum - ind
    padded = ((counts + _BT - 1) // _BT) * _BT
    pad_start = jnp.concatenate(
        [jnp.zeros((1,), jnp.int32), jnp.cumsum(padded).astype(jnp.int32)])
    starts = pad_start[:_E, None]
    pos0 = jnp.sum(jnp.where(oh_lo, starts + excl, 0), axis=0)
    pos1 = jnp.sum(jnp.where(oh_hi, starts + excl, 0), axis=0)
    tok = jnp.arange(_N, dtype=jnp.int32)
    base_ids = jnp.arange(_R, dtype=jnp.int32) % _N
    row_ids = base_ids.at[pos0].set(tok).at[pos1].set(tok)
    gate_pad = (jnp.zeros((_R,), jnp.float32)
                .at[pos0].set(g_lo).at[pos1].set(g_hi))
    tile_base = jnp.arange(_NT, dtype=jnp.int32) * _BT
    ends = pad_start[1:]
    eid = jnp.sum((ends[None, :] <= tile_base[:, None]).astype(jnp.int32),
                  axis=1)
    eid = jnp.minimum(eid, _E - 1)
    eid = jnp.where(tile_base < pad_start[_E], eid, -1).astype(jnp.int32)
    return row_ids, gate_pad, eid, pos0, pos1


def _gather_body(x_hbm, ids_hbm, out_hbm, idx_v, rows0, rows1, rows2, rows3,
                 sg0, sg1, sg2, sg3, ss0, ss1, ss2, ss3):
    wid = lax.axis_index("s") * _NC + lax.axis_index("c")
    pltpu.sync_copy(ids_hbm.at[wid], idx_v)
    bufs = (rows0, rows1, rows2, rows3)
    sgs = (sg0, sg1, sg2, sg3)
    sss = (ss0, ss1, ss2, ss3)
    gets = [None] * _NCH
    puts = [None] * _NCH
    for j in range(_GNB - 1):
        gets[j] = pltpu.async_copy(x_hbm.at[idx_v[j, :]], bufs[j], sgs[j])
    for i in range(_NCH):
        b = i % _GNB
        gets[i].wait()
        puts[i] = pltpu.async_copy(
            bufs[b], out_hbm.at[pl.ds(wid * _ROWS_W + i * _GCH, _GCH)],
            sss[b])
        j = i + _GNB - 1
        if j < _NCH:
            jb = j % _GNB
            if j >= _GNB:
                puts[j - _GNB].wait()
            gets[j] = pltpu.async_copy(
                x_hbm.at[idx_v[j, :]], bufs[jb], sgs[jb])
    for i in range(_NCH - _GNB, _NCH):
        puts[i].wait()


def _add_rows(dst, src):
    def row(rr, carry):
        for u in range(_DM // 16):
            sl = pl.ds(u * 16, 16)
            dst[rr, sl] = dst[rr, sl] + src[rr, sl]
        return carry

    lax.fori_loop(0, _CCH, row, 0)


def _combine_body(yg_hbm, p0_hbm, p1_hbm, out_hbm, i0_v, i1_v,
                  r0a, r1a, r0b, r1b, sa0, sa1, sb0, sb1, so0, so1):
    wid = lax.axis_index("s") * _NC + lax.axis_index("c")
    pltpu.sync_copy(p0_hbm.at[wid], i0_v)
    pltpu.sync_copy(p1_hbm.at[wid], i1_v)
    r0s = (r0a, r0b)
    r1s = (r1a, r1b)
    s0s = (sa0, sb0)
    s1s = (sa1, sb1)
    sos = (so0, so1)
    g0 = [None] * _NCC
    g1 = [None] * _NCC
    puts = [None] * _NCC
    g0[0] = pltpu.async_copy(yg_hbm.at[i0_v.at[0]], r0a, sa0)
    g1[0] = pltpu.async_copy(yg_hbm.at[i1_v.at[0]], r1a, sa1)
    for c in range(_NCC):
        b = c % 2
        g0[c].wait()
        g1[c].wait()
        if c + 1 < _NCC:
            if c >= 1:
                puts[c - 1].wait()
            g0[c + 1] = pltpu.async_copy(
                yg_hbm.at[i0_v.at[c + 1]], r0s[1 - b], s0s[1 - b])
            g1[c + 1] = pltpu.async_copy(
                yg_hbm.at[i1_v.at[c + 1]], r1s[1 - b], s1s[1 - b])
        _add_rows(r0s[b], r1s[b])
        puts[c] = pltpu.async_copy(
            r0s[b], out_hbm.at[pl.ds(wid * _TOK_W + c * _CCH, _CCH)],
            sos[b])
    puts[_NCC - 2].wait()
    puts[_NCC - 1].wait()


_sc_calls_cache = None


def _sc_calls():
    global _sc_calls_cache
    if _sc_calls_cache is None:
        mesh = plsc.VectorSubcoreMesh(
            core_axis_name="c", subcore_axis_name="s", num_cores=_NC)
        gather = pl.kernel(
            _gather_body, mesh=mesh,
            out_type=jax.ShapeDtypeStruct((_R, _DM), jnp.float32),
            scratch_types=[
                pltpu.VMEM((_NCH, _GCH), jnp.int32),
            ] + [pltpu.VMEM((_GCH, _DM), jnp.float32)] * _GNB
              + [pltpu.SemaphoreType.DMA] * (2 * _GNB),
        )
        combine = pl.kernel(
            _combine_body, mesh=mesh,
            out_type=jax.ShapeDtypeStruct((_N, _DM), jnp.float32),
            scratch_types=[
                pltpu.VMEM((_NCC, _CCH), jnp.int32),
                pltpu.VMEM((_NCC, _CCH), jnp.int32),
                pltpu.VMEM((_CCH, _DM), jnp.float32),
                pltpu.VMEM((_CCH, _DM), jnp.float32),
                pltpu.VMEM((_CCH, _DM), jnp.float32),
                pltpu.VMEM((_CCH, _DM), jnp.float32),
            ] + [pltpu.SemaphoreType.DMA] * 6,
        )
        _sc_calls_cache = (gather, combine)
    return _sc_calls_cache


def _mlp_body(eid_ref, xg_ref, w1_ref, b1_ref, w2_ref, b2_ref, gate_ref,
              out_ref):
    t = pl.program_id(0)
    f = pl.program_id(1)
    valid = eid_ref[t] >= 0

    @pl.when(valid)
    def _():
        h = jnp.dot(xg_ref[...], w1_ref[0],
                    preferred_element_type=jnp.float32)
        h = jnp.maximum(h + b1_ref[0], 0.0)
        contrib = jnp.dot(h, w2_ref[0], preferred_element_type=jnp.float32)

        @pl.when(f == 0)
        def _():
            out_ref[...] = contrib

        @pl.when(f != 0)
        def _():
            out_ref[...] += contrib

        @pl.when(f == _NF - 1)
        def _():
            out_ref[...] = (out_ref[...] + b2_ref[0]) * gate_ref[...]


def _eix(e_ref, t):
    return jnp.maximum(e_ref[t], 0)


_mlp_call = pl.pallas_call(
    _mlp_body,
    grid_spec=pltpu.PrefetchScalarGridSpec(
        num_scalar_prefetch=1,
        grid=(_NT, _NF),
        in_specs=[
            pl.BlockSpec((_BT, _DM), lambda t, f, e: (t, 0)),
            pl.BlockSpec((1, _DM, _FB), lambda t, f, e: (_eix(e, t), 0, f)),
            pl.BlockSpec((1, 1, _FB), lambda t, f, e: (_eix(e, t), 0, f)),
            pl.BlockSpec((1, _FB, _DM), lambda t, f, e: (_eix(e, t), f, 0)),
            pl.BlockSpec((1, 1, _DM), lambda t, f, e: (_eix(e, t), 0, 0)),
            pl.BlockSpec((_BT, 1), lambda t, f, e: (t, 0)),
        ],
        out_specs=pl.BlockSpec((_BT, _DM), lambda t, f, e: (t, 0)),
    ),
    out_shape=jax.ShapeDtypeStruct((_R, _DM), jnp.float32),
)


def kernel(x, w_gate, W1, b1, W2, b2):
    _gather_rows, _combine = _sc_calls()
    row_ids, gate_pad, eid, pos0, pos1 = _route(x, w_gate)
    xg = _gather_rows(x, row_ids.reshape(_NW, _NCH, _GCH))
    yg = _mlp_call(
        eid, xg,
        W1, b1.reshape(_E, 1, _DFF), W2, b2.reshape(_E, 1, _DM),
        gate_pad.reshape(_R, 1))
    out = _combine(
        yg,
        pos0.reshape(_NW, _NCC, _CCH),
        pos1.reshape(_NW, _NCC, _CCH))
    return out
